# baseline (speedup 1.0000x reference)
import math
import functools

import jax
import jax.numpy as jnp
from jax import lax
from jax.experimental import pallas as pl
from jax.experimental.pallas import tpu as pltpu
from jax.experimental.pallas import tpu_sc as plsc

HEADS = 8
DIMS = 16
SAMPLES = 2
N = 1024
B = 2
T = 12
LA_K = 16
D = HEADS * DIMS
BT = B * T
S = int(SAMPLES * math.log(N, 2))

bf = jnp.bfloat16
f32 = jnp.float32
i32 = jnp.int32

_NW = 32
_ROWS = BT * HEADS * S
_RPW = _ROWS // _NW
_CH = 24
_NCH = _RPW // _CH


def _sc_gather_masks(idx1, idx2, mask1, mask2_big):
    mesh = plsc.VectorSubcoreMesh(core_axis_name="c", subcore_axis_name="s")

    @functools.partial(
        pl.kernel, mesh=mesh,
        out_type=[jax.ShapeDtypeStruct((_ROWS, N), f32),
                  jax.ShapeDtypeStruct((_ROWS, N), f32)],
        scratch_types=[pltpu.VMEM((_CH,), i32),
                       pltpu.VMEM((_CH, N), f32),
                       pltpu.SemaphoreType.DMA],
    )
    def gather(idx1_hbm, idx2_hbm, m1_hbm, m2_hbm, o1_hbm, o2_hbm,
               idx_v, rows_v, sem):
        wid = lax.axis_index("s") * 2 + lax.axis_index("c")
        base = wid * _RPW
        for c in range(_NCH):
            off = base + c * _CH
            pltpu.sync_copy(idx1_hbm.at[pl.ds(off, _CH)], idx_v)
            pltpu.async_copy(m1_hbm.at[idx_v], rows_v, sem).wait()
            pltpu.sync_copy(rows_v, o1_hbm.at[pl.ds(off, _CH)])
            pltpu.sync_copy(idx2_hbm.at[pl.ds(off, _CH)], idx_v)
            pltpu.async_copy(m2_hbm.at[idx_v], rows_v, sem).wait()
            pltpu.sync_copy(rows_v, o2_hbm.at[pl.ds(off, _CH)])

    return gather(idx1, idx2, mask1, mask2_big)


def _attn_body(q_ref, k_ref, v_ref, mtl_ref, m1_ref, m2_ref,
               is1_ref, is2_ref, ofcw_ref, ofcb_ref, lnw_ref, lnb_ref,
               ffw_ref, ffb_ref, out_ref):
    Qf = q_ref[0]
    Kf = k_ref[0]
    Vf = v_ref[0]
    mtl = mtl_ref[0]
    iota_n = lax.broadcasted_iota(i32, (N, D), 0)
    iota_s = lax.broadcasted_iota(i32, (S, N), 0)
    is1 = is1_ref[0, 0]
    is2 = is2_ref[0, 0]
    dn = (((1,), (1,)), ((), ()))

    rows = []
    for s in range(S):
        sel = iota_n == mtl[s:s + 1, :]
        rows.append(jnp.sum(jnp.where(sel, Qf, 0.0), axis=0, keepdims=True))
    QR = jnp.concatenate(rows, axis=0)

    outsT = []
    for h in range(HEADS):
        sl = slice(DIMS * h, DIMS * (h + 1))
        qk = lax.dot_general(QR[:, sl].astype(bf), Kf[:, sl].astype(bf),
                             dn, preferred_element_type=f32) * 0.25
        m1r = m1_ref[0, S * h:S * (h + 1), :]
        m2r = m2_ref[0, S * h:S * (h + 1), :]
        qk = jnp.where(is1 != 0, qk * m1r, qk)
        qk = jnp.where(is2 != 0, qk * m2r, qk)
        mx = jnp.max(qk, axis=1, keepdims=True)
        e = jnp.exp(qk - mx)
        attn = e / jnp.sum(e, axis=1, keepdims=True)
        bv = jnp.max(attn, axis=0, keepdims=True)
        cp = jnp.min(jnp.where(attn == bv, iota_s, S),
                     axis=0, keepdims=True)
        v20 = lax.dot_general(attn.astype(bf), Vf[:, sl].astype(bf),
                              (((1,), (0,)), ((), ())),
                              preferred_element_type=f32)
        v20T = jnp.transpose(v20)
        ohT = jnp.zeros((DIMS, N), f32)
        for s in range(S):
            ohT = jnp.where(cp == s, v20T[:, s:s + 1], ohT)
        outsT.append(ohT)
    val = jnp.transpose(jnp.concatenate(outsT, axis=0))

    val = lax.dot_general(val.astype(bf), ofcw_ref[...].astype(bf), dn,
                          preferred_element_type=f32) + ofcb_ref[...]
    mu = jnp.mean(val, axis=1, keepdims=True)
    var = jnp.mean((val - mu) ** 2, axis=1, keepdims=True)
    val = (val - mu) / jnp.sqrt(var + 1e-5) * lnw_ref[...] + lnb_ref[...]
    out_ref[0] = lax.dot_general(val.astype(bf), ffw_ref[...].astype(bf), dn,
                                 preferred_element_type=f32) + ffb_ref[...]


def _attn_stage(Qf, Kf, Vf, mtl, m1rows, m2rows, is1, is2,
                ofc_w, ofc_b, ln_w, ln_b, ff_w, ff_b):
    full2 = lambda s: pl.BlockSpec(s, lambda i: (0, 0))
    smem = pl.BlockSpec((1, 1), lambda i: (0, 0), memory_space=pltpu.SMEM)
    return pl.pallas_call(
        _attn_body,
        grid=(BT,),
        in_specs=[
            pl.BlockSpec((1, N, D), lambda i: (i, 0, 0)),
            pl.BlockSpec((1, N, D), lambda i: (i, 0, 0)),
            pl.BlockSpec((1, N, D), lambda i: (i, 0, 0)),
            pl.BlockSpec((1, S, D), lambda i: (i, 0, 0)),
            pl.BlockSpec((1, HEADS * S, N), lambda i: (i, 0, 0)),
            pl.BlockSpec((1, HEADS * S, N), lambda i: (i, 0, 0)),
            smem, smem,
            full2((D, D)), full2((1, D)), full2((1, D)), full2((1, D)),
            full2((D, D)), full2((1, D)),
        ],
        out_specs=pl.BlockSpec((1, N, D), lambda i: (i, 0, 0)),
        out_shape=jax.ShapeDtypeStruct((BT, N, D), f32),
    )(Qf, Kf, Vf, mtl, m1rows, m2rows, is1, is2,
      ofc_w, ofc_b.reshape(1, D), ln_w.reshape(1, D), ln_b.reshape(1, D),
      ff_w, ff_b.reshape(1, D))


def kernel(x, spa_eigvalue, spa_eigvec, tem_eigvalue, tem_eigvec,
           IsMask1, IsMask2, localadj, mask1, mask2_big,
           proj_w, proj_b, qfc_w, qfc_b, kfc_w, kfc_b, vfc_w, vfc_b,
           ofc_w, ofc_b, ln_w, ln_b, ff_w, ff_b):
    h = HEADS
    x_ = x + jnp.matmul(spa_eigvec, jnp.diag(spa_eigvalue)) \
           + jnp.matmul(tem_eigvec, jnp.diag(tem_eigvalue))
    Q = x_ @ qfc_w.T + qfc_b
    K = x_ @ kfc_w.T + kfc_b
    V = x_ @ vfc_w.T + vfc_b
    Qh = jnp.concatenate(jnp.split(Q, h, axis=-1), axis=0)
    Kh = jnp.concatenate(jnp.split(K, h, axis=-1), axis=0)
    K_sample = Kh[:, :, localadj, :]
    p = Qh[..., None, :] * K_sample
    Q_K_sample = p[..., 0]
    for i in range(1, DIMS):
        Q_K_sample = Q_K_sample + p[..., i]
    M = jnp.squeeze(Q_K_sample @ proj_w.T + proj_b, axis=-1)
    _, M_top = lax.top_k(M, S)

    mtop_bth = jnp.transpose(M_top.reshape(h, B, T, S),
                             (1, 2, 0, 3)).reshape(BT, h, S)
    mtl = jnp.repeat(jnp.swapaxes(mtop_bth, 1, 2), DIMS, axis=2)
    idx1 = mtop_bth.reshape(-1).astype(i32)
    idx2 = (mtop_bth + (jnp.arange(h, dtype=i32) * N)[None, :, None]
            ).reshape(-1).astype(i32)

    m1rows, m2rows = _sc_gather_masks(idx1, idx2, mask1, mask2_big)
    m1rows = m1rows.reshape(BT, h * S, N)
    m2rows = m2rows.reshape(BT, h * S, N)

    is1 = jnp.asarray(IsMask1, i32).reshape(1, 1)
    is2 = jnp.asarray(IsMask2, i32).reshape(1, 1)
    out = _attn_stage(Q.reshape(BT, N, D), K.reshape(BT, N, D),
                      V.reshape(BT, N, D), mtl, m1rows, m2rows, is1, is2,
                      ofc_w, ofc_b, ln_w, ln_b, ff_w, ff_b)
    return out.reshape(B, T, N, D)


# speedup vs baseline: 1.3473x; 1.3473x over previous
import math
import functools

import jax
import jax.numpy as jnp
from jax import lax
from jax.experimental import pallas as pl
from jax.experimental.pallas import tpu as pltpu
from jax.experimental.pallas import tpu_sc as plsc

HEADS = 8
DIMS = 16
SAMPLES = 2
N = 1024
B = 2
T = 12
LA_K = 16
D = HEADS * DIMS
BT = B * T
S = int(SAMPLES * math.log(N, 2))

bf = jnp.bfloat16
f32 = jnp.float32
i32 = jnp.int32

_NW = 32
_ROWS = BT * HEADS * S
_RPW = _ROWS // _NW
_CH = 24
_NCH = _RPW // _CH


def _sc_gather_masks(idx1, idx2, mask1, mask2_big):
    mesh = plsc.VectorSubcoreMesh(core_axis_name="c", subcore_axis_name="s")

    @functools.partial(
        pl.kernel, mesh=mesh,
        out_type=[jax.ShapeDtypeStruct((_ROWS, N), f32),
                  jax.ShapeDtypeStruct((_ROWS, N), f32)],
        scratch_types=[pltpu.VMEM((_CH,), i32),
                       pltpu.VMEM((_CH, N), f32),
                       pltpu.SemaphoreType.DMA],
    )
    def gather(idx1_hbm, idx2_hbm, m1_hbm, m2_hbm, o1_hbm, o2_hbm,
               idx_v, rows_v, sem):
        wid = lax.axis_index("s") * 2 + lax.axis_index("c")
        base = wid * _RPW
        for c in range(_NCH):
            off = base + c * _CH
            pltpu.sync_copy(idx1_hbm.at[pl.ds(off, _CH)], idx_v)
            pltpu.async_copy(m1_hbm.at[idx_v], rows_v, sem).wait()
            pltpu.sync_copy(rows_v, o1_hbm.at[pl.ds(off, _CH)])
            pltpu.sync_copy(idx2_hbm.at[pl.ds(off, _CH)], idx_v)
            pltpu.async_copy(m2_hbm.at[idx_v], rows_v, sem).wait()
            pltpu.sync_copy(rows_v, o2_hbm.at[pl.ds(off, _CH)])

    return gather(idx1, idx2, mask1, mask2_big)


def _topk_body(m_ref, out_ref):
    work = m_ref[...]
    iota = lax.broadcasted_iota(i32, work.shape, 1)
    for s in range(S):
        mx = jnp.max(work, axis=1, keepdims=True)
        eq = work == mx
        idx = jnp.min(jnp.where(eq, iota, N), axis=1, keepdims=True)
        out_ref[:, s:s + 1] = idx
        work = jnp.where(iota == idx, -jnp.inf, work)


def _topk_idx(m):
    rows = m.shape[0]
    blk = 24
    return pl.pallas_call(
        _topk_body,
        grid=(rows // blk,),
        in_specs=[pl.BlockSpec((blk, N), lambda i: (i, 0))],
        out_specs=pl.BlockSpec((blk, S), lambda i: (i, 0)),
        out_shape=jax.ShapeDtypeStruct((rows, S), i32),
    )(m)


def _attn_body(q_ref, k_ref, v_ref, mtl_ref, m1_ref, m2_ref,
               is1_ref, is2_ref, ofcw_ref, ofcb_ref, lnw_ref, lnb_ref,
               ffw_ref, ffb_ref, out_ref):
    Qf = q_ref[0]
    Kf = k_ref[0]
    Vf = v_ref[0]
    mtl = mtl_ref[0]
    iota_n = lax.broadcasted_iota(i32, (N, D), 0)
    iota_s = lax.broadcasted_iota(i32, (S, N), 0)
    is1 = is1_ref[0, 0]
    is2 = is2_ref[0, 0]
    dn = (((1,), (1,)), ((), ()))

    rows = []
    for s in range(S):
        sel = iota_n == mtl[s:s + 1, :]
        rows.append(jnp.sum(jnp.where(sel, Qf, 0.0), axis=0, keepdims=True))
    QR = jnp.concatenate(rows, axis=0)

    outsT = []
    for h in range(HEADS):
        sl = slice(DIMS * h, DIMS * (h + 1))
        qk = lax.dot_general(QR[:, sl].astype(bf), Kf[:, sl].astype(bf),
                             dn, preferred_element_type=f32) * 0.25
        m1r = m1_ref[0, S * h:S * (h + 1), :]
        m2r = m2_ref[0, S * h:S * (h + 1), :]
        qk = jnp.where(is1 != 0, qk * m1r, qk)
        qk = jnp.where(is2 != 0, qk * m2r, qk)
        mx = jnp.max(qk, axis=1, keepdims=True)
        e = jnp.exp(qk - mx)
        attn = e / jnp.sum(e, axis=1, keepdims=True)
        bv = jnp.max(attn, axis=0, keepdims=True)
        cp = jnp.min(jnp.where(attn == bv, iota_s, S),
                     axis=0, keepdims=True)
        v20 = lax.dot_general(attn.astype(bf), Vf[:, sl].astype(bf),
                              (((1,), (0,)), ((), ())),
                              preferred_element_type=f32)
        v20T = jnp.transpose(v20)
        ohT = jnp.zeros((DIMS, N), f32)
        for s in range(S):
            ohT = jnp.where(cp == s, v20T[:, s:s + 1], ohT)
        outsT.append(ohT)
    val = jnp.transpose(jnp.concatenate(outsT, axis=0))

    val = lax.dot_general(val.astype(bf), ofcw_ref[...].astype(bf), dn,
                          preferred_element_type=f32) + ofcb_ref[...]
    mu = jnp.mean(val, axis=1, keepdims=True)
    var = jnp.mean((val - mu) ** 2, axis=1, keepdims=True)
    val = (val - mu) / jnp.sqrt(var + 1e-5) * lnw_ref[...] + lnb_ref[...]
    out_ref[0] = lax.dot_general(val.astype(bf), ffw_ref[...].astype(bf), dn,
                                 preferred_element_type=f32) + ffb_ref[...]


def _attn_stage(Qf, Kf, Vf, mtl, m1rows, m2rows, is1, is2,
                ofc_w, ofc_b, ln_w, ln_b, ff_w, ff_b):
    full2 = lambda s: pl.BlockSpec(s, lambda i: (0, 0))
    smem = pl.BlockSpec((1, 1), lambda i: (0, 0), memory_space=pltpu.SMEM)
    return pl.pallas_call(
        _attn_body,
        grid=(BT,),
        in_specs=[
            pl.BlockSpec((1, N, D), lambda i: (i, 0, 0)),
            pl.BlockSpec((1, N, D), lambda i: (i, 0, 0)),
            pl.BlockSpec((1, N, D), lambda i: (i, 0, 0)),
            pl.BlockSpec((1, S, D), lambda i: (i, 0, 0)),
            pl.BlockSpec((1, HEADS * S, N), lambda i: (i, 0, 0)),
            pl.BlockSpec((1, HEADS * S, N), lambda i: (i, 0, 0)),
            smem, smem,
            full2((D, D)), full2((1, D)), full2((1, D)), full2((1, D)),
            full2((D, D)), full2((1, D)),
        ],
        out_specs=pl.BlockSpec((1, N, D), lambda i: (i, 0, 0)),
        out_shape=jax.ShapeDtypeStruct((BT, N, D), f32),
    )(Qf, Kf, Vf, mtl, m1rows, m2rows, is1, is2,
      ofc_w, ofc_b.reshape(1, D), ln_w.reshape(1, D), ln_b.reshape(1, D),
      ff_w, ff_b.reshape(1, D))


def kernel(x, spa_eigvalue, spa_eigvec, tem_eigvalue, tem_eigvec,
           IsMask1, IsMask2, localadj, mask1, mask2_big,
           proj_w, proj_b, qfc_w, qfc_b, kfc_w, kfc_b, vfc_w, vfc_b,
           ofc_w, ofc_b, ln_w, ln_b, ff_w, ff_b):
    h = HEADS
    x_ = x + jnp.matmul(spa_eigvec, jnp.diag(spa_eigvalue)) \
           + jnp.matmul(tem_eigvec, jnp.diag(tem_eigvalue))
    Q = x_ @ qfc_w.T + qfc_b
    K = x_ @ kfc_w.T + kfc_b
    V = x_ @ vfc_w.T + vfc_b
    Qh = jnp.concatenate(jnp.split(Q, h, axis=-1), axis=0)
    Kh = jnp.concatenate(jnp.split(K, h, axis=-1), axis=0)
    K_sample = Kh[:, :, localadj, :]
    Q_K_sample = jnp.squeeze(
        jnp.matmul(Qh[..., None, :], jnp.swapaxes(K_sample, -2, -1)), axis=-2)
    M = jnp.squeeze(Q_K_sample @ proj_w.T + proj_b, axis=-1)
    M_top = _topk_idx(M.reshape(HEADS * B * T, N)).reshape(HEADS * B, T, S)

    mtop_bth = jnp.transpose(M_top.reshape(h, B, T, S),
                             (1, 2, 0, 3)).reshape(BT, h, S)
    mtl = jnp.repeat(jnp.swapaxes(mtop_bth, 1, 2), DIMS, axis=2)
    idx1 = mtop_bth.reshape(-1).astype(i32)
    idx2 = (mtop_bth + (jnp.arange(h, dtype=i32) * N)[None, :, None]
            ).reshape(-1).astype(i32)

    m1rows, m2rows = _sc_gather_masks(idx1, idx2, mask1, mask2_big)
    m1rows = m1rows.reshape(BT, h * S, N)
    m2rows = m2rows.reshape(BT, h * S, N)

    is1 = jnp.asarray(IsMask1, i32).reshape(1, 1)
    is2 = jnp.asarray(IsMask2, i32).reshape(1, 1)
    out = _attn_stage(Q.reshape(BT, N, D), K.reshape(BT, N, D),
                      V.reshape(BT, N, D), mtl, m1rows, m2rows, is1, is2,
                      ofc_w, ofc_b, ln_w, ln_b, ff_w, ff_b)
    return out.reshape(B, T, N, D)
